# baseline (speedup 1.0000x reference)
import jax
import jax.numpy as jnp
from jax.experimental import pallas as pl
from jax.experimental.pallas import tpu as pltpu

N, E = 2048, 4096
DV, DE, NHID, NCLUST = 128, 16, 64, 10
ALPHA = 0.2

BM = 256
BE = 256

_CPARAMS = pltpu.CompilerParams(
    dimension_semantics=("arbitrary",),
    vmem_limit_bytes=110 * 1024 * 1024,
)


def _node_adjusted_kernel(T_ref, He_ref, p_ref, adj_ref, out_ref, colmax_ref):
    i = pl.program_id(0)
    d = jax.lax.dot_general(p_ref[...], He_ref[...], (((1,), (1,)), ((), ())),
                            preferred_element_type=jnp.float32)
    Trow = T_ref[pl.ds(i * BM, BM), :]
    mult = jax.lax.dot_general(Trow * d, T_ref[...],
                               (((1,), (1,)), ((), ())),
                               preferred_element_type=jnp.float32)
    rows = i * BM + jax.lax.broadcasted_iota(jnp.int32, (BM, N), 0)
    cols = jax.lax.broadcasted_iota(jnp.int32, (BM, N), 1)
    adj = adj_ref[...]
    adjusted = jnp.where(rows == cols, adj, mult * adj)
    out_ref[...] = adjusted
    bmax = jnp.max(adjusted, axis=0, keepdims=True)

    @pl.when(i == 0)
    def _():
        colmax_ref[...] = bmax

    @pl.when(i != 0)
    def _():
        colmax_ref[...] = jnp.maximum(colmax_ref[...], bmax)


def _edge_adjusted_kernel(T_ref, Hv_ref, p_ref, eadj_ref, out_ref, colmax_ref):
    i = pl.program_id(0)
    d = jax.lax.dot_general(Hv_ref[...], p_ref[...], (((1,), (1,)), ((), ())),
                            preferred_element_type=jnp.float32)
    Tcol = T_ref[:, pl.ds(i * BE, BE)]
    mult = jax.lax.dot_general(Tcol * d, T_ref[...],
                               (((0,), (0,)), ((), ())),
                               preferred_element_type=jnp.float32)
    rows = i * BE + jax.lax.broadcasted_iota(jnp.int32, (BE, E), 0)
    cols = jax.lax.broadcasted_iota(jnp.int32, (BE, E), 1)
    eadj = eadj_ref[...]
    adjusted = jnp.where(rows == cols, eadj, mult * eadj)
    out_ref[...] = adjusted
    bmax = jnp.max(adjusted, axis=0, keepdims=True)

    @pl.when(i == 0)
    def _():
        colmax_ref[...] = bmax

    @pl.when(i != 0)
    def _():
        colmax_ref[...] = jnp.maximum(colmax_ref[...], bmax)


def _apply_kernel(adj_ref, H_ref, W_ref, b_ref, colmax_ref, out_ref):
    X = jax.lax.dot_general(H_ref[...], W_ref[...], (((1,), (0,)), ((), ())),
                            preferred_element_type=jnp.float32)
    normalized = adj_ref[...] * (1.0 / colmax_ref[...])
    out_ref[...] = jax.lax.dot_general(
        normalized, X, (((1,), (0,)), ((), ())),
        preferred_element_type=jnp.float32) + b_ref[...]


def _apply_q_kernel(adj_ref, H_ref, W_ref, b_ref, colmax_ref, mu_ref,
                    x_ref, q_ref):
    X = jax.lax.dot_general(H_ref[...], W_ref[...], (((1,), (0,)), ((), ())),
                            preferred_element_type=jnp.float32)
    normalized = adj_ref[...] * (1.0 / colmax_ref[...])
    x = jax.lax.dot_general(normalized, X, (((1,), (0,)), ((), ())),
                            preferred_element_type=jnp.float32) + b_ref[...]
    x_ref[...] = x
    mu = mu_ref[...]
    x2 = jnp.sum(x * x, axis=1, keepdims=True)
    mu2 = jnp.sum(mu * mu, axis=1)[None, :]
    cross = jax.lax.dot_general(x, mu, (((1,), (1,)), ((), ())),
                                preferred_element_type=jnp.float32)
    dist = x2 - 2.0 * cross + mu2
    q = 1.0 / (1.0 + dist / ALPHA + 1e-8)
    q = q ** (ALPHA + 1.0) / 2.0
    q_ref[...] = q / jnp.sum(q, axis=1, keepdims=True)


def kernel(features, edge_features, adj, edge_adj, Tmat,
           W1, b1, p1, W2, b2, p2, W3, b3, p3, mu):
    f32 = jnp.float32

    def full(shape):
        return pl.BlockSpec(shape, lambda i: (0,) * len(shape))

    adjusted1, colmax1 = pl.pallas_call(
        _node_adjusted_kernel,
        grid=(N // BM,),
        in_specs=[full((N, E)), full((E, DE)), full((1, DE)),
                  pl.BlockSpec((BM, N), lambda i: (i, 0))],
        out_specs=[pl.BlockSpec((BM, N), lambda i: (i, 0)), full((1, N))],
        out_shape=[jax.ShapeDtypeStruct((N, N), f32),
                   jax.ShapeDtypeStruct((1, N), f32)],
        compiler_params=_CPARAMS,
    )(Tmat, edge_features, p1, adj)

    Xh1 = pl.pallas_call(
        _apply_kernel,
        grid=(N // BM,),
        in_specs=[pl.BlockSpec((BM, N), lambda i: (i, 0)),
                  full((N, DV)), full((DV, NHID)), full((1, NHID)),
                  full((1, N))],
        out_specs=pl.BlockSpec((BM, NHID), lambda i: (i, 0)),
        out_shape=jax.ShapeDtypeStruct((N, NHID), f32),
        compiler_params=_CPARAMS,
    )(adjusted1, features, W1, b1.reshape(1, NHID), colmax1)

    adjusted2, colmax2 = pl.pallas_call(
        _edge_adjusted_kernel,
        grid=(E // BE,),
        in_specs=[full((N, E)), full((N, NHID)), full((1, NHID)),
                  pl.BlockSpec((BE, E), lambda i: (i, 0))],
        out_specs=[pl.BlockSpec((BE, E), lambda i: (i, 0)), full((1, E))],
        out_shape=[jax.ShapeDtypeStruct((E, E), f32),
                   jax.ShapeDtypeStruct((1, E), f32)],
        compiler_params=_CPARAMS,
    )(Tmat, Xh1, p2, edge_adj)

    Zh = pl.pallas_call(
        _apply_kernel,
        grid=(E // BE,),
        in_specs=[pl.BlockSpec((BE, E), lambda i: (i, 0)),
                  full((E, DE)), full((DE, DE)), full((1, DE)),
                  full((1, E))],
        out_specs=pl.BlockSpec((BE, DE), lambda i: (i, 0)),
        out_shape=jax.ShapeDtypeStruct((E, DE), f32),
        compiler_params=_CPARAMS,
    )(adjusted2, edge_features, W2, b2.reshape(1, DE), colmax2)

    adjusted3, colmax3 = pl.pallas_call(
        _node_adjusted_kernel,
        grid=(N // BM,),
        in_specs=[full((N, E)), full((E, DE)), full((1, DE)),
                  pl.BlockSpec((BM, N), lambda i: (i, 0))],
        out_specs=[pl.BlockSpec((BM, N), lambda i: (i, 0)), full((1, N))],
        out_shape=[jax.ShapeDtypeStruct((N, N), f32),
                   jax.ShapeDtypeStruct((1, N), f32)],
        compiler_params=_CPARAMS,
    )(Tmat, Zh, p3, adj)

    x, q = pl.pallas_call(
        _apply_q_kernel,
        grid=(N // BM,),
        in_specs=[pl.BlockSpec((BM, N), lambda i: (i, 0)),
                  full((N, NHID)), full((NHID, DV)), full((1, DV)),
                  full((1, N)), full((NCLUST, DV))],
        out_specs=[pl.BlockSpec((BM, DV), lambda i: (i, 0)),
                   pl.BlockSpec((BM, NCLUST), lambda i: (i, 0))],
        out_shape=[jax.ShapeDtypeStruct((N, DV), f32),
                   jax.ShapeDtypeStruct((N, NCLUST), f32)],
        compiler_params=_CPARAMS,
    )(adjusted3, Xh1, W3, b3.reshape(1, DV), colmax3, mu)

    return (x, q)


# speedup vs baseline: 1.0218x; 1.0218x over previous
import jax
import jax.numpy as jnp
from jax.experimental import pallas as pl
from jax.experimental.pallas import tpu as pltpu

N, E = 2048, 4096
DV, DE, NHID, NCLUST = 128, 16, 64, 10
ALPHA = 0.2

BM = 256
BE = 256

_CPARAMS = pltpu.CompilerParams(
    dimension_semantics=("arbitrary",),
    vmem_limit_bytes=110 * 1024 * 1024,
)


def _node_adjusted_kernel(T_ref, He_ref, p_ref, adj_ref, out_ref, colmax_ref):
    i = pl.program_id(0)
    d = jax.lax.dot_general(p_ref[...], He_ref[...], (((1,), (1,)), ((), ())),
                            preferred_element_type=jnp.float32)
    db = d.astype(jnp.bfloat16)
    Trow = T_ref[pl.ds(i * BM, BM), :]
    mult = jax.lax.dot_general(Trow * db, T_ref[...],
                               (((1,), (1,)), ((), ())),
                               preferred_element_type=jnp.float32)
    rows = i * BM + jax.lax.broadcasted_iota(jnp.int32, (BM, N), 0)
    cols = jax.lax.broadcasted_iota(jnp.int32, (BM, N), 1)
    adj = adj_ref[...]
    adjusted = jnp.where(rows == cols, adj, mult * adj)
    out_ref[...] = adjusted
    bmax = jnp.max(adjusted, axis=0, keepdims=True)

    @pl.when(i == 0)
    def _():
        colmax_ref[...] = bmax

    @pl.when(i != 0)
    def _():
        colmax_ref[...] = jnp.maximum(colmax_ref[...], bmax)


def _edge_adjusted_kernel(T_ref, Hv_ref, p_ref, eadj_ref, out_ref, colmax_ref):
    i = pl.program_id(0)
    d = jax.lax.dot_general(Hv_ref[...], p_ref[...], (((1,), (1,)), ((), ())),
                            preferred_element_type=jnp.float32)
    db = d.astype(jnp.bfloat16)
    Tcol = T_ref[:, pl.ds(i * BE, BE)]
    mult = jax.lax.dot_general(Tcol * db, T_ref[...],
                               (((0,), (0,)), ((), ())),
                               preferred_element_type=jnp.float32)
    rows = i * BE + jax.lax.broadcasted_iota(jnp.int32, (BE, E), 0)
    cols = jax.lax.broadcasted_iota(jnp.int32, (BE, E), 1)
    eadj = eadj_ref[...]
    adjusted = jnp.where(rows == cols, eadj, mult * eadj)
    out_ref[...] = adjusted
    bmax = jnp.max(adjusted, axis=0, keepdims=True)

    @pl.when(i == 0)
    def _():
        colmax_ref[...] = bmax

    @pl.when(i != 0)
    def _():
        colmax_ref[...] = jnp.maximum(colmax_ref[...], bmax)


def _apply_kernel(adj_ref, H_ref, W_ref, b_ref, colmax_ref, out_ref):
    X = jax.lax.dot_general(H_ref[...], W_ref[...], (((1,), (0,)), ((), ())),
                            preferred_element_type=jnp.float32)
    normalized = adj_ref[...] * (1.0 / colmax_ref[...])
    out_ref[...] = jax.lax.dot_general(
        normalized, X, (((1,), (0,)), ((), ())),
        preferred_element_type=jnp.float32) + b_ref[...]


def _apply_q_kernel(adj_ref, H_ref, W_ref, b_ref, colmax_ref, mu_ref,
                    x_ref, q_ref):
    X = jax.lax.dot_general(H_ref[...], W_ref[...], (((1,), (0,)), ((), ())),
                            preferred_element_type=jnp.float32)
    normalized = adj_ref[...] * (1.0 / colmax_ref[...])
    x = jax.lax.dot_general(normalized, X, (((1,), (0,)), ((), ())),
                            preferred_element_type=jnp.float32) + b_ref[...]
    x_ref[...] = x
    mu = mu_ref[...]
    x2 = jnp.sum(x * x, axis=1, keepdims=True)
    mu2 = jnp.sum(mu * mu, axis=1)[None, :]
    cross = jax.lax.dot_general(x, mu, (((1,), (1,)), ((), ())),
                                preferred_element_type=jnp.float32)
    dist = x2 - 2.0 * cross + mu2
    q = 1.0 / (1.0 + dist / ALPHA + 1e-8)
    q = q ** (ALPHA + 1.0) / 2.0
    q_ref[...] = q / jnp.sum(q, axis=1, keepdims=True)


def kernel(features, edge_features, adj, edge_adj, Tmat,
           W1, b1, p1, W2, b2, p2, W3, b3, p3, mu):
    f32 = jnp.float32
    Tbf = Tmat.astype(jnp.bfloat16)

    def full(shape):
        return pl.BlockSpec(shape, lambda i: (0,) * len(shape))

    adjusted1, colmax1 = pl.pallas_call(
        _node_adjusted_kernel,
        grid=(N // BM,),
        in_specs=[full((N, E)), full((E, DE)), full((1, DE)),
                  pl.BlockSpec((BM, N), lambda i: (i, 0))],
        out_specs=[pl.BlockSpec((BM, N), lambda i: (i, 0)), full((1, N))],
        out_shape=[jax.ShapeDtypeStruct((N, N), f32),
                   jax.ShapeDtypeStruct((1, N), f32)],
        compiler_params=_CPARAMS,
    )(Tbf, edge_features, p1, adj)

    Xh1 = pl.pallas_call(
        _apply_kernel,
        grid=(N // BM,),
        in_specs=[pl.BlockSpec((BM, N), lambda i: (i, 0)),
                  full((N, DV)), full((DV, NHID)), full((1, NHID)),
                  full((1, N))],
        out_specs=pl.BlockSpec((BM, NHID), lambda i: (i, 0)),
        out_shape=jax.ShapeDtypeStruct((N, NHID), f32),
        compiler_params=_CPARAMS,
    )(adjusted1, features, W1, b1.reshape(1, NHID), colmax1)

    adjusted2, colmax2 = pl.pallas_call(
        _edge_adjusted_kernel,
        grid=(E // BE,),
        in_specs=[full((N, E)), full((N, NHID)), full((1, NHID)),
                  pl.BlockSpec((BE, E), lambda i: (i, 0))],
        out_specs=[pl.BlockSpec((BE, E), lambda i: (i, 0)), full((1, E))],
        out_shape=[jax.ShapeDtypeStruct((E, E), f32),
                   jax.ShapeDtypeStruct((1, E), f32)],
        compiler_params=_CPARAMS,
    )(Tbf, Xh1, p2, edge_adj)

    Zh = pl.pallas_call(
        _apply_kernel,
        grid=(E // BE,),
        in_specs=[pl.BlockSpec((BE, E), lambda i: (i, 0)),
                  full((E, DE)), full((DE, DE)), full((1, DE)),
                  full((1, E))],
        out_specs=pl.BlockSpec((BE, DE), lambda i: (i, 0)),
        out_shape=jax.ShapeDtypeStruct((E, DE), f32),
        compiler_params=_CPARAMS,
    )(adjusted2, edge_features, W2, b2.reshape(1, DE), colmax2)

    adjusted3, colmax3 = pl.pallas_call(
        _node_adjusted_kernel,
        grid=(N // BM,),
        in_specs=[full((N, E)), full((E, DE)), full((1, DE)),
                  pl.BlockSpec((BM, N), lambda i: (i, 0))],
        out_specs=[pl.BlockSpec((BM, N), lambda i: (i, 0)), full((1, N))],
        out_shape=[jax.ShapeDtypeStruct((N, N), f32),
                   jax.ShapeDtypeStruct((1, N), f32)],
        compiler_params=_CPARAMS,
    )(Tbf, Zh, p3, adj)

    x, q = pl.pallas_call(
        _apply_q_kernel,
        grid=(N // BM,),
        in_specs=[pl.BlockSpec((BM, N), lambda i: (i, 0)),
                  full((N, NHID)), full((NHID, DV)), full((1, DV)),
                  full((1, N)), full((NCLUST, DV))],
        out_specs=[pl.BlockSpec((BM, DV), lambda i: (i, 0)),
                   pl.BlockSpec((BM, NCLUST), lambda i: (i, 0))],
        out_shape=[jax.ShapeDtypeStruct((N, DV), f32),
                   jax.ShapeDtypeStruct((N, NCLUST), f32)],
        compiler_params=_CPARAMS,
    )(adjusted3, Xh1, W3, b3.reshape(1, DV), colmax3, mu)

    return (x, q)


# speedup vs baseline: 1.1142x; 1.0905x over previous
import jax
import jax.numpy as jnp
from jax.experimental import pallas as pl
from jax.experimental.pallas import tpu as pltpu

N, E = 2048, 4096
DV, DE, NHID, NCLUST = 128, 16, 64, 10
ALPHA = 0.2

BM = 256
BE = 256
NB = N // BM
EB = E // BE

_CPARAMS = pltpu.CompilerParams(
    dimension_semantics=("arbitrary",),
    vmem_limit_bytes=110 * 1024 * 1024,
)


def _node_layer_kernel(T_ref, He_ref, p_ref, adj_ref, Hv_ref, W_ref, b_ref,
                       out_ref, adj_scr, colmax_scr):
    i = pl.program_id(0)

    @pl.when(i < NB)
    def _():
        d = jax.lax.dot_general(p_ref[...], He_ref[...],
                                (((1,), (1,)), ((), ())),
                                preferred_element_type=jnp.float32)
        db = d.astype(jnp.bfloat16)
        Trow = T_ref[pl.ds(i * BM, BM), :]
        mult = jax.lax.dot_general(Trow * db, T_ref[...],
                                   (((1,), (1,)), ((), ())),
                                   preferred_element_type=jnp.float32)
        rows = i * BM + jax.lax.broadcasted_iota(jnp.int32, (BM, N), 0)
        cols = jax.lax.broadcasted_iota(jnp.int32, (BM, N), 1)
        adjusted = jnp.where(rows == cols, adj_ref[...], mult * adj_ref[...])
        adj_scr[pl.ds(i * BM, BM), :] = adjusted
        bmax = jnp.max(adjusted, axis=0, keepdims=True)

        @pl.when(i == 0)
        def _():
            colmax_scr[...] = bmax

        @pl.when(i != 0)
        def _():
            colmax_scr[...] = jnp.maximum(colmax_scr[...], bmax)

    @pl.when(i >= NB)
    def _():
        j = i - NB
        X = jax.lax.dot_general(Hv_ref[...], W_ref[...],
                                (((1,), (0,)), ((), ())),
                                preferred_element_type=jnp.float32)
        blk = adj_scr[pl.ds(j * BM, BM), :] * (1.0 / colmax_scr[...])
        out_ref[...] = jax.lax.dot_general(
            blk, X, (((1,), (0,)), ((), ())),
            preferred_element_type=jnp.float32) + b_ref[...]


def _edge_layer_kernel(T_ref, Hv_ref, p_ref, eadj_ref, He_ref, W_ref, b_ref,
                       out_ref, adj_scr, colmax_scr):
    i = pl.program_id(0)

    @pl.when(i < EB)
    def _():
        d = jax.lax.dot_general(Hv_ref[...], p_ref[...],
                                (((1,), (1,)), ((), ())),
                                preferred_element_type=jnp.float32)
        db = d.astype(jnp.bfloat16)
        Tcol = T_ref[:, pl.ds(i * BE, BE)]
        mult = jax.lax.dot_general(Tcol * db, T_ref[...],
                                   (((0,), (0,)), ((), ())),
                                   preferred_element_type=jnp.float32)
        rows = i * BE + jax.lax.broadcasted_iota(jnp.int32, (BE, E), 0)
        cols = jax.lax.broadcasted_iota(jnp.int32, (BE, E), 1)
        adjusted = jnp.where(rows == cols, eadj_ref[...],
                             mult * eadj_ref[...])
        adj_scr[pl.ds(i * BE, BE), :] = adjusted.astype(jnp.bfloat16)
        bmax = jnp.max(adjusted, axis=0, keepdims=True)

        @pl.when(i == 0)
        def _():
            colmax_scr[...] = bmax

        @pl.when(i != 0)
        def _():
            colmax_scr[...] = jnp.maximum(colmax_scr[...], bmax)

    @pl.when(i >= EB)
    def _():
        j = i - EB
        X = jax.lax.dot_general(He_ref[...], W_ref[...],
                                (((1,), (0,)), ((), ())),
                                preferred_element_type=jnp.float32)
        blk = adj_scr[pl.ds(j * BE, BE), :].astype(jnp.float32)
        blk = blk * (1.0 / colmax_scr[...])
        out_ref[...] = jax.lax.dot_general(
            blk, X, (((1,), (0,)), ((), ())),
            preferred_element_type=jnp.float32) + b_ref[...]


def _node_layer_q_kernel(T_ref, He_ref, p_ref, adj_ref, Hv_ref, W_ref, b_ref,
                         mu_ref, x_ref, q_ref, adj_scr, colmax_scr):
    i = pl.program_id(0)

    @pl.when(i < NB)
    def _():
        d = jax.lax.dot_general(p_ref[...], He_ref[...],
                                (((1,), (1,)), ((), ())),
                                preferred_element_type=jnp.float32)
        db = d.astype(jnp.bfloat16)
        Trow = T_ref[pl.ds(i * BM, BM), :]
        mult = jax.lax.dot_general(Trow * db, T_ref[...],
                                   (((1,), (1,)), ((), ())),
                                   preferred_element_type=jnp.float32)
        rows = i * BM + jax.lax.broadcasted_iota(jnp.int32, (BM, N), 0)
        cols = jax.lax.broadcasted_iota(jnp.int32, (BM, N), 1)
        adjusted = jnp.where(rows == cols, adj_ref[...], mult * adj_ref[...])
        adj_scr[pl.ds(i * BM, BM), :] = adjusted
        bmax = jnp.max(adjusted, axis=0, keepdims=True)

        @pl.when(i == 0)
        def _():
            colmax_scr[...] = bmax

        @pl.when(i != 0)
        def _():
            colmax_scr[...] = jnp.maximum(colmax_scr[...], bmax)

    @pl.when(i >= NB)
    def _():
        j = i - NB
        X = jax.lax.dot_general(Hv_ref[...], W_ref[...],
                                (((1,), (0,)), ((), ())),
                                preferred_element_type=jnp.float32)
        blk = adj_scr[pl.ds(j * BM, BM), :] * (1.0 / colmax_scr[...])
        x = jax.lax.dot_general(blk, X, (((1,), (0,)), ((), ())),
                                preferred_element_type=jnp.float32) + b_ref[...]
        x_ref[...] = x
        mu = mu_ref[...]
        x2 = jnp.sum(x * x, axis=1, keepdims=True)
        mu2 = jnp.sum(mu * mu, axis=1)[None, :]
        cross = jax.lax.dot_general(x, mu, (((1,), (1,)), ((), ())),
                                    preferred_element_type=jnp.float32)
        dist = x2 - 2.0 * cross + mu2
        q = 1.0 / (1.0 + dist / ALPHA + 1e-8)
        q = q ** (ALPHA + 1.0) / 2.0
        q_ref[...] = q / jnp.sum(q, axis=1, keepdims=True)


def kernel(features, edge_features, adj, edge_adj, Tmat,
           W1, b1, p1, W2, b2, p2, W3, b3, p3, mu):
    f32 = jnp.float32
    bf16 = jnp.bfloat16
    Tbf = Tmat.astype(bf16)

    def full(shape):
        return pl.BlockSpec(shape, lambda i: (0,) * len(shape))

    Xh1 = pl.pallas_call(
        _node_layer_kernel,
        grid=(2 * NB,),
        in_specs=[full((N, E)), full((E, DE)), full((1, DE)),
                  pl.BlockSpec((BM, N), lambda i: (jnp.minimum(i, NB - 1), 0)),
                  full((N, DV)), full((DV, NHID)), full((1, NHID))],
        out_specs=pl.BlockSpec((BM, NHID),
                               lambda i: (jnp.maximum(i - NB, 0), 0)),
        out_shape=jax.ShapeDtypeStruct((N, NHID), f32),
        scratch_shapes=[pltpu.VMEM((N, N), f32), pltpu.VMEM((1, N), f32)],
        compiler_params=_CPARAMS,
    )(Tbf, edge_features, p1, adj, features, W1, b1.reshape(1, NHID))

    Zh = pl.pallas_call(
        _edge_layer_kernel,
        grid=(2 * EB,),
        in_specs=[full((N, E)), full((N, NHID)), full((1, NHID)),
                  pl.BlockSpec((BE, E), lambda i: (jnp.minimum(i, EB - 1), 0)),
                  full((E, DE)), full((DE, DE)), full((1, DE))],
        out_specs=pl.BlockSpec((BE, DE),
                               lambda i: (jnp.maximum(i - EB, 0), 0)),
        out_shape=jax.ShapeDtypeStruct((E, DE), f32),
        scratch_shapes=[pltpu.VMEM((E, E), bf16), pltpu.VMEM((1, E), f32)],
        compiler_params=_CPARAMS,
    )(Tbf, Xh1, p2, edge_adj, edge_features, W2, b2.reshape(1, DE))

    x, q = pl.pallas_call(
        _node_layer_q_kernel,
        grid=(2 * NB,),
        in_specs=[full((N, E)), full((E, DE)), full((1, DE)),
                  pl.BlockSpec((BM, N), lambda i: (jnp.minimum(i, NB - 1), 0)),
                  full((N, NHID)), full((NHID, DV)), full((1, DV)),
                  full((NCLUST, DV))],
        out_specs=[pl.BlockSpec((BM, DV),
                                lambda i: (jnp.maximum(i - NB, 0), 0)),
                   pl.BlockSpec((BM, NCLUST),
                                lambda i: (jnp.maximum(i - NB, 0), 0))],
        out_shape=[jax.ShapeDtypeStruct((N, DV), f32),
                   jax.ShapeDtypeStruct((N, NCLUST), f32)],
        scratch_shapes=[pltpu.VMEM((N, N), f32), pltpu.VMEM((1, N), f32)],
        compiler_params=_CPARAMS,
    )(Tbf, Zh, p3, adj, Xh1, W3, b3.reshape(1, DV), mu)

    return (x, q)
